# baseline (speedup 1.0000x reference)
import jax
import jax.numpy as jnp
from jax.experimental import pallas as pl
from jax.experimental.pallas import tpu as pltpu

_BATCH = 16
_BAGS_LEN = 1042
_CT = 961
_GUIDE = _BAGS_LEN - _CT
_D = 768
_K = 3
_ITERS = 50
_CLS = 3


def _bag_kernel(xp_ref, xpt_ref, g_ref, xsq_ref, w_ref, b_ref, o_ref):
    xp = xp_ref[0]
    xpt = xpt_ref[0]
    guide = g_ref[0]
    x_sq = xsq_ref[0]

    def dist_assign(centers):
        prod = jax.lax.dot_general(
            centers, xpt, (((1,), (0,)), ((), ())),
            preferred_element_type=jnp.float32)
        sq = centers * centers
        acc = sq[:, 0:128]
        for k in range(1, _D // 128):
            acc = acc + sq[:, 128 * k:128 * (k + 1)]
        c_sq = jnp.sum(acc, axis=1, keepdims=True)
        d = (x_sq - 2.0 * prod) + c_sq
        d0, d1, d2 = d[0:1, :], d[1:2, :], d[2:3, :]
        return jnp.where(
            (d0 <= d1) & (d0 <= d2), 0,
            jnp.where(d1 <= d2, 1, 2)).astype(jnp.int32)

    def body(carry):
        i, centers, assign, _ = carry
        new_assign = dist_assign(centers)
        onehot = (new_assign == jax.lax.broadcasted_iota(
            jnp.int32, (_K, _CT), 0)).astype(jnp.float32)
        sums = jax.lax.dot_general(
            onehot, xp, (((1,), (0,)), ((), ())),
            preferred_element_type=jnp.float32,
            precision=jax.lax.Precision.HIGHEST)
        cnts = jnp.sum(onehot, axis=1, keepdims=True)
        centers = sums / jnp.maximum(cnts, 1.0)
        changed = jnp.any(new_assign != assign)
        return (i + 1, centers, new_assign, changed)

    def cond(carry):
        i, _, _, changed = carry
        return jnp.logical_and(i < _ITERS, changed)

    init = (jnp.int32(0), xp[:_K, :],
            jnp.full((1, _CT), -1, jnp.int32), jnp.bool_(True))
    _, centers, _, _ = jax.lax.while_loop(cond, body, init)
    assign = dist_assign(centers)

    scores = jax.lax.dot_general(
        guide, xpt, (((1,), (0,)), ((), ())),
        preferred_element_type=jnp.float32) * jnp.float32(1.0 / (_D ** 0.5))
    pvec = jnp.zeros((1, _CT), jnp.float32)
    for c in range(2):
        masked = jnp.where(assign == c, scores, jnp.float32(-1e9))
        m = jnp.max(masked, axis=1, keepdims=True)
        e = jnp.exp(masked - m)
        s = jnp.sum(e, axis=1, keepdims=True)
        pvec = pvec + jnp.sum(e / s, axis=0, keepdims=True)
    pvec = pvec * jnp.float32(1.0 / _GUIDE)

    agg = jax.lax.dot_general(
        pvec, xp, (((1,), (0,)), ((), ())),
        preferred_element_type=jnp.float32)
    logits = jax.lax.dot_general(
        agg, w_ref[...], (((1,), (0,)), ((), ())),
        preferred_element_type=jnp.float32) + b_ref[...]
    o_ref[0] = 2.0 * logits


def kernel(x, W, b):
    y = x.reshape(_BATCH, _BAGS_LEN, _D)
    xp = y[:, :_CT, :]
    xpt = jnp.swapaxes(xp, 1, 2)
    guide = y[:, _CT:, :]
    x_sq = jnp.sum(xp * xp, axis=2)[:, None, :]
    b2 = b.reshape(1, _CLS)

    out = pl.pallas_call(
        _bag_kernel,
        grid=(_BATCH,),
        in_specs=[
            pl.BlockSpec((1, _CT, _D), lambda i: (i, 0, 0)),
            pl.BlockSpec((1, _D, _CT), lambda i: (i, 0, 0)),
            pl.BlockSpec((1, _GUIDE, _D), lambda i: (i, 0, 0)),
            pl.BlockSpec((1, 1, _CT), lambda i: (i, 0, 0)),
            pl.BlockSpec((_D, _CLS), lambda i: (0, 0)),
            pl.BlockSpec((1, _CLS), lambda i: (0, 0)),
        ],
        out_specs=pl.BlockSpec((1, 1, _CLS), lambda i: (i, 0, 0)),
        out_shape=jax.ShapeDtypeStruct((_BATCH, 1, _CLS), jnp.float32),
        compiler_params=pltpu.CompilerParams(
            dimension_semantics=("parallel",)),
    )(xp, xpt, guide, x_sq, W, b2)
    return out.reshape(_BATCH, _CLS)


# speedup vs baseline: 1.0008x; 1.0008x over previous
import jax
import jax.numpy as jnp
from jax.experimental import pallas as pl
from jax.experimental.pallas import tpu as pltpu

_BATCH = 16
_BAGS_LEN = 1042
_CT = 961
_GUIDE = _BAGS_LEN - _CT
_D = 768
_K = 3
_ITERS = 50
_CLS = 3


def _bag_kernel(xp_ref, xpt_ref, g_ref, xsq_ref, w_ref, b_ref, o_ref):
    xp = xp_ref[0]
    xpt = xpt_ref[0]
    guide = g_ref[0]
    x_sq = xsq_ref[0]

    def dist_assign(centers):
        prod = jax.lax.dot_general(
            centers, xpt, (((1,), (0,)), ((), ())),
            preferred_element_type=jnp.float32)
        sq = centers * centers
        acc = sq[:, 0:128]
        for k in range(1, _D // 128):
            acc = acc + sq[:, 128 * k:128 * (k + 1)]
        c_sq = jnp.sum(acc, axis=1, keepdims=True)
        d = (x_sq - 2.0 * prod) + c_sq
        d0, d1, d2 = d[0:1, :], d[1:2, :], d[2:3, :]
        return jnp.where(
            (d0 <= d1) & (d0 <= d2), 0,
            jnp.where(d1 <= d2, 1, 2)).astype(jnp.int32)

    def body(carry):
        i, centers, assign, _ = carry
        new_assign = dist_assign(centers)
        onehot = (new_assign == jax.lax.broadcasted_iota(
            jnp.int32, (_K, _CT), 0)).astype(jnp.float32)
        sums = jax.lax.dot_general(
            onehot, xp, (((1,), (0,)), ((), ())),
            preferred_element_type=jnp.float32,
            precision=jax.lax.Precision.HIGHEST)
        cnts = jnp.sum(onehot, axis=1, keepdims=True)
        centers = sums / jnp.maximum(cnts, 1.0)
        changed = jnp.any(new_assign != assign)
        return (i + 1, centers, new_assign, changed)

    def cond(carry):
        i, _, _, changed = carry
        return jnp.logical_and(i < _ITERS, changed)

    init = (jnp.int32(0), xp[:_K, :],
            jnp.full((1, _CT), -1, jnp.int32), jnp.bool_(True))
    _, centers, _, _ = jax.lax.while_loop(cond, body, init)
    assign = dist_assign(centers)

    scores = jax.lax.dot_general(
        guide, xpt, (((1,), (0,)), ((), ())),
        preferred_element_type=jnp.float32) * jnp.float32(1.0 / (_D ** 0.5))
    pvec = jnp.zeros((1, _CT), jnp.float32)
    for c in range(2):
        masked = jnp.where(assign == c, scores, jnp.float32(-1e9))
        m = jnp.max(masked, axis=1, keepdims=True)
        e = jnp.exp(masked - m)
        s = jnp.sum(e, axis=1, keepdims=True)
        pvec = pvec + jnp.sum(e / s, axis=0, keepdims=True)
    pvec = pvec * jnp.float32(1.0 / _GUIDE)

    agg = jax.lax.dot_general(
        pvec, xp, (((1,), (0,)), ((), ())),
        preferred_element_type=jnp.float32)
    logits = jax.lax.dot_general(
        agg, w_ref[...], (((1,), (0,)), ((), ())),
        preferred_element_type=jnp.float32) + b_ref[...]
    o_ref[0] = 2.0 * logits


def kernel(x, W, b):
    y = x.reshape(_BATCH, _BAGS_LEN, _D)
    xp = y[:, :_CT, :]
    xpt = jnp.swapaxes(xp, 1, 2)
    guide = y[:, _CT:, :]
    x_sq = jnp.sum(xp * xp, axis=2)[:, None, :]
    b2 = b.reshape(1, _CLS)

    out = pl.pallas_call(
        _bag_kernel,
        grid=(_BATCH,),
        in_specs=[
            pl.BlockSpec((1, _CT, _D), lambda i: (i, 0, 0)),
            pl.BlockSpec((1, _D, _CT), lambda i: (i, 0, 0)),
            pl.BlockSpec((1, _GUIDE, _D), lambda i: (i, 0, 0)),
            pl.BlockSpec((1, 1, _CT), lambda i: (i, 0, 0)),
            pl.BlockSpec((_D, _CLS), lambda i: (0, 0)),
            pl.BlockSpec((1, _CLS), lambda i: (0, 0)),
        ],
        out_specs=pl.BlockSpec((1, 1, _CLS), lambda i: (i, 0, 0)),
        out_shape=jax.ShapeDtypeStruct((_BATCH, 1, _CLS), jnp.float32),
        compiler_params=pltpu.CompilerParams(
            dimension_semantics=("arbitrary",)),
    )(xp, xpt, guide, x_sq, W, b2)
    return out.reshape(_BATCH, _CLS)
